# Initial kernel scaffold; baseline (speedup 1.0000x reference)
import jax
import jax.numpy as jnp
from jax.experimental import pallas as pl


def kernel(input, x_A, y_A, x_B, y_B, ordinal_relation):
    raise NotImplementedError("write your pallas kernel here")


# speedup vs baseline: 1.1899x; 1.1899x over previous
import functools

import jax
import jax.numpy as jnp
from jax import lax
from jax.experimental import pallas as pl
from jax.experimental.pallas import tpu as pltpu
from jax.experimental.pallas import tpu_sc as plsc

B, P, H, W = 8, 100000, 512, 512
HW = H * W
BP = B * P
NC, NS, LANES = 2, 16, 16
NW = NC * NS
PT = BP // NW
TPB = P // PT

_mesh = plsc.VectorSubcoreMesh(core_axis_name="c", subcore_axis_name="s")


@functools.partial(
    pl.kernel,
    mesh=_mesh,
    out_type=(
        jax.ShapeDtypeStruct((BP,), jnp.float32),
        jax.ShapeDtypeStruct((BP,), jnp.float32),
    ),
    scratch_types=[
        pltpu.VMEM((PT,), jnp.int32),
        pltpu.VMEM((PT,), jnp.int32),
        pltpu.VMEM((PT,), jnp.int32),
        pltpu.VMEM((PT,), jnp.float32),
        pltpu.VMEM((PT,), jnp.float32),
        pltpu.SemaphoreType.DMA,
    ],
)
def _sc_gather(depth, xa, ya, xb, yb, out_a, out_b, buf_x, buf_y, buf_i,
               z_a, z_b, sem):
    c = lax.axis_index("c")
    s = lax.axis_index("s")
    w = c * NS + s
    base = pl.multiple_of(w * PT, 8)
    boff = (w // TPB) * HW

    def idx_step(o):
        o = pl.multiple_of(o, 8)
        buf_i[pl.ds(o, LANES)] = (
            buf_y[pl.ds(o, LANES)] * W + buf_x[pl.ds(o, LANES)] + boff
        )

    def compute_indices():
        def body(i, carry):
            idx_step(i * LANES)
            return carry
        lax.fori_loop(0, PT // LANES, body, 0)
        idx_step(PT - LANES)

    pltpu.sync_copy(xa.at[pl.ds(base, PT)], buf_x)
    pltpu.sync_copy(ya.at[pl.ds(base, PT)], buf_y)
    compute_indices()
    cp_a = pltpu.async_copy(depth.at[buf_i], z_a, sem)
    pltpu.sync_copy(xb.at[pl.ds(base, PT)], buf_x)
    pltpu.sync_copy(yb.at[pl.ds(base, PT)], buf_y)
    cp_a.wait()
    compute_indices()
    cp_b = pltpu.async_copy(depth.at[buf_i], z_b, sem)
    pltpu.sync_copy(z_a, out_a.at[pl.ds(base, PT)])
    cp_b.wait()
    pltpu.sync_copy(z_b, out_b.at[pl.ds(base, PT)])


def _loss_body(za_ref, zb_ref, g_ref, o_ref):
    d = za_ref[...] - zb_ref[...]
    g = g_ref[...]
    mask = jnp.abs(g)
    loss = mask * jnp.log(1.0 + jnp.exp(-g * d)) + (1.0 - mask) * (d * d)
    o_ref[0, 0] = jnp.sum(loss) / BP


_ROWS = BP // 128

_loss_call = pl.pallas_call(
    _loss_body,
    out_shape=jax.ShapeDtypeStruct((1, 1), jnp.float32),
    out_specs=pl.BlockSpec(memory_space=pltpu.SMEM),
)


def kernel(input, x_A, y_A, x_B, y_B, ordinal_relation):
    depth = input.reshape(B * H * W)
    z_a, z_b = _sc_gather(
        depth,
        x_A.reshape(BP),
        y_A.reshape(BP),
        x_B.reshape(BP),
        y_B.reshape(BP),
    )
    out = _loss_call(
        z_a.reshape(_ROWS, 128),
        z_b.reshape(_ROWS, 128),
        ordinal_relation.reshape(_ROWS, 128),
    )
    return out.reshape(1)


# speedup vs baseline: 1.7141x; 1.4405x over previous
import functools

import jax
import jax.numpy as jnp
from jax import lax
from jax.experimental import pallas as pl
from jax.experimental.pallas import tpu as pltpu
from jax.experimental.pallas import tpu_sc as plsc

B, P, H, W = 8, 100000, 512, 512
HW = H * W
BP = B * P
NC, NS = 2, 16
ROWS = BP // 128
SP2 = 2 * HW
SL2 = SP2 // NS
PPP = 2 * P
CNT_LO, CNT_HI = 12504, 12496

_mesh = plsc.VectorSubcoreMesh(core_axis_name="c", subcore_axis_name="s")


def _idx_body(xa_ref, ya_ref, xb_ref, yb_ref, ia_ref, ib_ref):
    r = lax.broadcasted_iota(jnp.int32, (ROWS, 128), 0)
    lane = lax.broadcasted_iota(jnp.int32, (ROWS, 128), 1)
    g = r * 128 + lane
    base = ((g // P) % 2) * HW
    ia_ref[...] = base + ya_ref[...] * W + xa_ref[...]
    ib_ref[...] = base + yb_ref[...] * W + xb_ref[...]


_idx_call = pl.pallas_call(
    _idx_body,
    out_shape=(
        jax.ShapeDtypeStruct((ROWS, 128), jnp.int32),
        jax.ShapeDtypeStruct((ROWS, 128), jnp.int32),
    ),
)


@functools.partial(
    pl.kernel,
    mesh=_mesh,
    out_type=(
        jax.ShapeDtypeStruct((BP,), jnp.float32),
        jax.ShapeDtypeStruct((BP,), jnp.float32),
    ),
    scratch_types=[
        pltpu.VMEM((CNT_LO,), jnp.int32),
        pltpu.VMEM((CNT_LO,), jnp.int32),
        pltpu.VMEM((CNT_LO,), jnp.float32),
        pltpu.VMEM((CNT_LO,), jnp.float32),
        pltpu.VMEM_SHARED((SP2,), jnp.float32),
        pltpu.SemaphoreType.DMA,
        pltpu.SemaphoreType.DMA,
        pltpu.SemaphoreType.DMA,
    ],
)
def _sc_gather(depth, idx_a, idx_b, out_a, out_b, via, vib, z_a, z_b,
               spmem, sem_s, sem_a, sem_b):
    c = lax.axis_index("c")
    s = lax.axis_index("s")

    def stage(k):
        dbase = (4 * c + 2 * k) * HW + s * SL2
        return pltpu.async_copy(
            depth.at[pl.ds(pl.multiple_of(dbase, 8), SL2)],
            spmem.at[pl.ds(s * SL2, SL2)], sem_s)

    def gathers(k):
        def side(cnt, off_s):
            gb = pl.multiple_of(c * (2 * PPP) + k * PPP + off_s, 8)
            ia = via.at[pl.ds(0, cnt)]
            ib = vib.at[pl.ds(0, cnt)]
            za = z_a.at[pl.ds(0, cnt)]
            zb = z_b.at[pl.ds(0, cnt)]
            pltpu.sync_copy(idx_a.at[pl.ds(gb, cnt)], ia)
            pltpu.sync_copy(idx_b.at[pl.ds(gb, cnt)], ib)
            cp_a = pltpu.async_copy(spmem.at[ia], za, sem_a)
            cp_b = pltpu.async_copy(spmem.at[ib], zb, sem_b)
            cp_a.wait()
            pltpu.sync_copy(za, out_a.at[pl.ds(gb, cnt)])
            cp_b.wait()
            pltpu.sync_copy(zb, out_b.at[pl.ds(gb, cnt)])
        pl.when(s < 8)(lambda: side(CNT_LO, s * CNT_LO))
        pl.when(s >= 8)(lambda: side(CNT_HI, 8 * CNT_LO + (s - 8) * CNT_HI))

    stage(0).wait()
    plsc.subcore_barrier()
    gathers(0)
    plsc.subcore_barrier()
    stage(1).wait()
    plsc.subcore_barrier()
    gathers(1)


def _loss_body(za_ref, zb_ref, g_ref, o_ref):
    d = za_ref[...] - zb_ref[...]
    g = g_ref[...]
    mask = jnp.abs(g)
    loss = mask * jnp.log(1.0 + jnp.exp(-g * d)) + (1.0 - mask) * (d * d)
    o_ref[0, 0] = jnp.sum(loss) / BP


_loss_call = pl.pallas_call(
    _loss_body,
    out_shape=jax.ShapeDtypeStruct((1, 1), jnp.float32),
    out_specs=pl.BlockSpec(memory_space=pltpu.SMEM),
)


def kernel(input, x_A, y_A, x_B, y_B, ordinal_relation):
    depth = input.reshape(B * H * W)
    ia, ib = _idx_call(
        x_A.reshape(ROWS, 128),
        y_A.reshape(ROWS, 128),
        x_B.reshape(ROWS, 128),
        y_B.reshape(ROWS, 128),
    )
    z_a, z_b = _sc_gather(depth, ia.reshape(BP), ib.reshape(BP))
    out = _loss_call(
        z_a.reshape(ROWS, 128),
        z_b.reshape(ROWS, 128),
        ordinal_relation.reshape(ROWS, 128),
    )
    return out.reshape(1)


# speedup vs baseline: 2.0176x; 1.1771x over previous
import functools

import jax
import jax.numpy as jnp
from jax import lax
from jax.experimental import pallas as pl
from jax.experimental.pallas import tpu as pltpu
from jax.experimental.pallas import tpu_sc as plsc

B, P, H, W = 8, 100000, 512, 512
HW = H * W
BP = B * P
NC, NS = 2, 16
ROWS = BP // 128
SP2 = 2 * HW
SL2 = SP2 // NS
PPP = 2 * P
CNT_LO, CNT_HI = 12504, 12496

_mesh = plsc.VectorSubcoreMesh(core_axis_name="c", subcore_axis_name="s")


def _idx_body(xa_ref, ya_ref, xb_ref, yb_ref, ia_ref, ib_ref):
    b = lax.broadcasted_iota(jnp.int32, (B, P), 0)
    base = (b % 2) * HW
    ia_ref[...] = base + ya_ref[...] * W + xa_ref[...]
    ib_ref[...] = base + yb_ref[...] * W + xb_ref[...]


_idx_call = pl.pallas_call(
    _idx_body,
    out_shape=(
        jax.ShapeDtypeStruct((B, P), jnp.int32),
        jax.ShapeDtypeStruct((B, P), jnp.int32),
    ),
)


@functools.partial(
    pl.kernel,
    mesh=_mesh,
    out_type=(
        jax.ShapeDtypeStruct((BP,), jnp.float32),
        jax.ShapeDtypeStruct((BP,), jnp.float32),
    ),
    scratch_types=[
        pltpu.VMEM((CNT_LO,), jnp.int32),
        pltpu.VMEM((CNT_LO,), jnp.int32),
        pltpu.VMEM((CNT_LO,), jnp.float32),
        pltpu.VMEM((CNT_LO,), jnp.float32),
        pltpu.VMEM_SHARED((SP2,), jnp.float32),
        pltpu.SemaphoreType.DMA,
        pltpu.SemaphoreType.DMA,
        pltpu.SemaphoreType.DMA,
    ],
)
def _sc_gather(depth, idx_a, idx_b, out_a, out_b, via, vib, z_a, z_b,
               spmem, sem_s, sem_a, sem_b):
    c = lax.axis_index("c")
    s = lax.axis_index("s")

    def stage(k):
        dbase = (4 * c + 2 * k) * HW + s * SL2
        return pltpu.async_copy(
            depth.at[pl.ds(pl.multiple_of(dbase, 8), SL2)],
            spmem.at[pl.ds(s * SL2, SL2)], sem_s)

    def gathers(k):
        def side(cnt, off_s):
            gb = pl.multiple_of(c * (2 * PPP) + k * PPP + off_s, 8)
            ia = via.at[pl.ds(0, cnt)]
            ib = vib.at[pl.ds(0, cnt)]
            za = z_a.at[pl.ds(0, cnt)]
            zb = z_b.at[pl.ds(0, cnt)]
            pltpu.sync_copy(idx_a.at[pl.ds(gb, cnt)], ia)
            pltpu.sync_copy(idx_b.at[pl.ds(gb, cnt)], ib)
            cp_a = pltpu.async_copy(spmem.at[ia], za, sem_a)
            cp_b = pltpu.async_copy(spmem.at[ib], zb, sem_b)
            cp_a.wait()
            pltpu.sync_copy(za, out_a.at[pl.ds(gb, cnt)])
            cp_b.wait()
            pltpu.sync_copy(zb, out_b.at[pl.ds(gb, cnt)])
        pl.when(s < 8)(lambda: side(CNT_LO, s * CNT_LO))
        pl.when(s >= 8)(lambda: side(CNT_HI, 8 * CNT_LO + (s - 8) * CNT_HI))

    stage(0).wait()
    plsc.subcore_barrier()
    gathers(0)
    plsc.subcore_barrier()
    stage(1).wait()
    plsc.subcore_barrier()
    gathers(1)


def _loss_body(za_ref, zb_ref, g_ref, o_ref):
    d = za_ref[...] - zb_ref[...]
    g = g_ref[...]
    mask = jnp.abs(g)
    loss = mask * jnp.log(1.0 + jnp.exp(-g * d)) + (1.0 - mask) * (d * d)
    o_ref[0, 0] = jnp.sum(loss) / BP


_loss_call = pl.pallas_call(
    _loss_body,
    out_shape=jax.ShapeDtypeStruct((1, 1), jnp.float32),
    out_specs=pl.BlockSpec(memory_space=pltpu.SMEM),
)


def kernel(input, x_A, y_A, x_B, y_B, ordinal_relation):
    depth = input.reshape(B * H * W)
    ia, ib = _idx_call(x_A, y_A, x_B, y_B)
    z_a, z_b = _sc_gather(depth, ia.reshape(BP), ib.reshape(BP))
    out = _loss_call(
        z_a.reshape(ROWS, 128),
        z_b.reshape(ROWS, 128),
        ordinal_relation.reshape(ROWS, 128),
    )
    return out.reshape(1)


# speedup vs baseline: 2.1823x; 1.0816x over previous
import functools

import jax
import jax.numpy as jnp
from jax import lax
from jax.experimental import pallas as pl
from jax.experimental.pallas import tpu as pltpu
from jax.experimental.pallas import tpu_sc as plsc

B, P, H, W = 8, 100000, 512, 512
HW = H * W
BP = B * P
PMAIN = 99968
PREM = P - PMAIN
PADP = 100096
BPP = B * PADP
ROWS_P = BPP // 128
NC, NS = 2, 16
SP2 = 2 * HW
SL2 = SP2 // NS
CNT = 2 * PADP // NS

_mesh = plsc.VectorSubcoreMesh(core_axis_name="c", subcore_axis_name="s")


def _idx_body(xam, xar, yam, yar, xbm, xbr, ybm, ybr, om, orr,
              ia_ref, ib_ref, op_ref):
    lane = lax.broadcasted_iota(jnp.int32, (1, 128), 1).reshape(128)
    valid = lane < PREM

    def side(xm, ym, xr, yr):
        main = ym[...] * W + xm[...]
        remv = yr[...] * W + xr[...]
        pieces = []
        for b in range(B):
            base = (b % 2) * HW
            pieces.append((main[b : b + 1] + base).reshape(PMAIN))
            pieces.append(jnp.where(
                valid, remv[b : b + 1].reshape(128) + base, 0))
        return jnp.concatenate(pieces, axis=0)

    ia_ref[...] = side(xam, yam, xar, yar)
    ib_ref[...] = side(xbm, ybm, xbr, ybr)
    opieces = []
    for b in range(B):
        opieces.append(om[b : b + 1].reshape(PMAIN))
        opieces.append(jnp.where(valid, orr[b : b + 1].reshape(128), 0.0))
    op_ref[...] = jnp.concatenate(opieces, axis=0)


def _in_pair():
    return (
        pl.BlockSpec((B, PMAIN), lambda i: (0, 0)),
        pl.BlockSpec((B, 128), lambda i: (0, PMAIN // 128)),
    )


_idx_call = pl.pallas_call(
    _idx_body,
    grid=(1,),
    out_specs=(
        pl.BlockSpec((BPP,), lambda i: (0,)),
        pl.BlockSpec((BPP,), lambda i: (0,)),
        pl.BlockSpec((BPP,), lambda i: (0,)),
    ),
    in_specs=[
        *_in_pair(),
        *_in_pair(),
        *_in_pair(),
        *_in_pair(),
        *_in_pair(),
    ],
    out_shape=(
        jax.ShapeDtypeStruct((BPP,), jnp.int32),
        jax.ShapeDtypeStruct((BPP,), jnp.int32),
        jax.ShapeDtypeStruct((BPP,), jnp.float32),
    ),
)


@functools.partial(
    pl.kernel,
    mesh=_mesh,
    out_type=(
        jax.ShapeDtypeStruct((BPP,), jnp.float32),
        jax.ShapeDtypeStruct((BPP,), jnp.float32),
    ),
    scratch_types=[
        pltpu.VMEM((CNT,), jnp.int32),
        pltpu.VMEM((CNT,), jnp.int32),
        pltpu.VMEM((CNT,), jnp.float32),
        pltpu.VMEM((CNT,), jnp.float32),
        pltpu.VMEM_SHARED((SP2,), jnp.float32),
        pltpu.SemaphoreType.DMA,
        pltpu.SemaphoreType.DMA,
        pltpu.SemaphoreType.DMA,
    ],
)
def _sc_gather(depth, idx_a, idx_b, out_a, out_b, via, vib, z_a, z_b,
               spmem, sem_s, sem_a, sem_b):
    c = lax.axis_index("c")
    s = lax.axis_index("s")

    def stage(k):
        dbase = (4 * c + 2 * k) * HW + s * SL2
        return pltpu.async_copy(
            depth.at[pl.ds(pl.multiple_of(dbase, 8), SL2)],
            spmem.at[pl.ds(s * SL2, SL2)], sem_s)

    def gathers(k):
        gb = pl.multiple_of((4 * c + 2 * k) * PADP + s * CNT, 8)
        pltpu.sync_copy(idx_a.at[pl.ds(gb, CNT)], via)
        pltpu.sync_copy(idx_b.at[pl.ds(gb, CNT)], vib)
        cp_a = pltpu.async_copy(spmem.at[via], z_a, sem_a)
        cp_b = pltpu.async_copy(spmem.at[vib], z_b, sem_b)
        cp_a.wait()
        pltpu.sync_copy(z_a, out_a.at[pl.ds(gb, CNT)])
        cp_b.wait()
        pltpu.sync_copy(z_b, out_b.at[pl.ds(gb, CNT)])

    stage(0).wait()
    plsc.subcore_barrier()
    gathers(0)
    plsc.subcore_barrier()
    stage(1).wait()
    plsc.subcore_barrier()
    gathers(1)


def _loss_body(za_ref, zb_ref, g_ref, o_ref):
    d = za_ref[...] - zb_ref[...]
    g = g_ref[...]
    mask = jnp.abs(g)
    loss = mask * jnp.log(1.0 + jnp.exp(-g * d)) + (1.0 - mask) * (d * d)
    o_ref[0, 0] = jnp.sum(loss) / BP


_loss_call = pl.pallas_call(
    _loss_body,
    out_shape=jax.ShapeDtypeStruct((1, 1), jnp.float32),
    out_specs=pl.BlockSpec(memory_space=pltpu.SMEM),
)


def kernel(input, x_A, y_A, x_B, y_B, ordinal_relation):
    depth = input.reshape(B * H * W)
    ia, ib, ordp = _idx_call(x_A, x_A, y_A, y_A, x_B, x_B, y_B, y_B,
                             ordinal_relation, ordinal_relation)
    z_a, z_b = _sc_gather(depth, ia, ib)
    out = _loss_call(
        z_a.reshape(ROWS_P, 128),
        z_b.reshape(ROWS_P, 128),
        ordp.reshape(ROWS_P, 128),
    )
    return out.reshape(1)


# speedup vs baseline: 2.4584x; 1.1265x over previous
import functools

import jax
import jax.numpy as jnp
from jax import lax
from jax.experimental import pallas as pl
from jax.experimental.pallas import tpu as pltpu
from jax.experimental.pallas import tpu_sc as plsc

B, P, H, W = 8, 100000, 512, 512
HW = H * W
BP = B * P
PMAIN = 99968
PREM = P - PMAIN
PADP = 100096
BPP = B * PADP
ROWS_P = BPP // 128
NC, NS = 2, 16
SLI = HW // NS
CNT = PADP // NS

_mesh = plsc.VectorSubcoreMesh(core_axis_name="c", subcore_axis_name="s")


def _idx_body(xam, xar, yam, yar, xbm, xbr, ybm, ybr, om, orr,
              ia_ref, ib_ref, op_ref):
    lane = lax.broadcasted_iota(jnp.int32, (1, 128), 1).reshape(128)
    valid = lane < PREM

    def side(xm, ym, xr, yr):
        main = ym[...] * W + xm[...]
        remv = yr[...] * W + xr[...]
        pieces = []
        for b in range(B):
            base = (b % 2) * HW
            pieces.append((main[b : b + 1] + base).reshape(PMAIN))
            pieces.append(jnp.where(
                valid, remv[b : b + 1].reshape(128) + base, 0))
        return jnp.concatenate(pieces, axis=0)

    ia_ref[...] = side(xam, yam, xar, yar)
    ib_ref[...] = side(xbm, ybm, xbr, ybr)
    opieces = []
    for b in range(B):
        opieces.append(om[b : b + 1].reshape(PMAIN))
        opieces.append(jnp.where(valid, orr[b : b + 1].reshape(128), 0.0))
    op_ref[...] = jnp.concatenate(opieces, axis=0)


def _in_pair():
    return (
        pl.BlockSpec((B, PMAIN), lambda i: (0, 0)),
        pl.BlockSpec((B, 128), lambda i: (0, PMAIN // 128)),
    )


_idx_call = pl.pallas_call(
    _idx_body,
    grid=(1,),
    out_specs=(
        pl.BlockSpec((BPP,), lambda i: (0,)),
        pl.BlockSpec((BPP,), lambda i: (0,)),
        pl.BlockSpec((BPP,), lambda i: (0,)),
    ),
    in_specs=[
        *_in_pair(),
        *_in_pair(),
        *_in_pair(),
        *_in_pair(),
        *_in_pair(),
    ],
    out_shape=(
        jax.ShapeDtypeStruct((BPP,), jnp.int32),
        jax.ShapeDtypeStruct((BPP,), jnp.int32),
        jax.ShapeDtypeStruct((BPP,), jnp.float32),
    ),
)


@functools.partial(
    pl.kernel,
    mesh=_mesh,
    out_type=(
        jax.ShapeDtypeStruct((BPP,), jnp.float32),
        jax.ShapeDtypeStruct((BPP,), jnp.float32),
    ),
    scratch_types=[
        pltpu.VMEM((CNT,), jnp.int32),
        pltpu.VMEM((CNT,), jnp.int32),
        pltpu.VMEM((CNT,), jnp.int32),
        pltpu.VMEM((CNT,), jnp.int32),
        pltpu.VMEM((CNT,), jnp.float32),
        pltpu.VMEM((CNT,), jnp.float32),
        pltpu.VMEM_SHARED((HW,), jnp.float32),
        pltpu.VMEM_SHARED((HW,), jnp.float32),
        pltpu.SemaphoreType.DMA,
        pltpu.SemaphoreType.DMA,
        pltpu.SemaphoreType.DMA,
    ],
)
def _sc_gather(depth, idx_a, idx_b, out_a, out_b, via0, vib0, via1, vib1,
               z_a, z_b, slab0, slab1, sem_s, sem_a, sem_b):
    c = lax.axis_index("c")
    s = lax.axis_index("s")
    slabs = (slab0, slab1)
    bufs = ((via0, vib0), (via1, vib1))

    def stage(i):
        dbase = (4 * c + i) * HW + s * SLI
        return pltpu.async_copy(
            depth.at[pl.ds(pl.multiple_of(dbase, 8), SLI)],
            slabs[i % 2].at[pl.ds(s * SLI, SLI)], sem_s)

    def ldidx(i):
        gb = pl.multiple_of((4 * c + i) * PADP + s * CNT, 8)
        va, vb = bufs[i % 2]
        pltpu.sync_copy(idx_a.at[pl.ds(gb, CNT)], va)
        pltpu.sync_copy(idx_b.at[pl.ds(gb, CNT)], vb)

    st = stage(0)
    ldidx(0)
    st.wait()
    plsc.subcore_barrier()
    for i in range(4):
        if i < 3:
            st = stage(i + 1)
        va, vb = bufs[i % 2]
        cp_a = pltpu.async_copy(slabs[i % 2].at[va], z_a, sem_a)
        cp_b = pltpu.async_copy(slabs[i % 2].at[vb], z_b, sem_b)
        if i < 3:
            ldidx(i + 1)
        gb = pl.multiple_of((4 * c + i) * PADP + s * CNT, 8)
        cp_a.wait()
        pltpu.sync_copy(z_a, out_a.at[pl.ds(gb, CNT)])
        cp_b.wait()
        pltpu.sync_copy(z_b, out_b.at[pl.ds(gb, CNT)])
        if i < 3:
            st.wait()
            plsc.subcore_barrier()


def _loss_body(za_ref, zb_ref, g_ref, o_ref):
    d = za_ref[...] - zb_ref[...]
    g = g_ref[...]
    mask = jnp.abs(g)
    loss = mask * jnp.log(1.0 + jnp.exp(-g * d)) + (1.0 - mask) * (d * d)
    o_ref[0, 0] = jnp.sum(loss) / BP


_loss_call = pl.pallas_call(
    _loss_body,
    out_shape=jax.ShapeDtypeStruct((1, 1), jnp.float32),
    out_specs=pl.BlockSpec(memory_space=pltpu.SMEM),
)


def kernel(input, x_A, y_A, x_B, y_B, ordinal_relation):
    depth = input.reshape(B * H * W)
    ia, ib, ordp = _idx_call(x_A, x_A, y_A, y_A, x_B, x_B, y_B, y_B,
                             ordinal_relation, ordinal_relation)
    z_a, z_b = _sc_gather(depth, ia, ib)
    out = _loss_call(
        z_a.reshape(ROWS_P, 128),
        z_b.reshape(ROWS_P, 128),
        ordp.reshape(ROWS_P, 128),
    )
    return out.reshape(1)


# speedup vs baseline: 2.4972x; 1.0158x over previous
import functools

import jax
import jax.numpy as jnp
from jax import lax
from jax.experimental import pallas as pl
from jax.experimental.pallas import tpu as pltpu
from jax.experimental.pallas import tpu_sc as plsc

B, P, H, W = 8, 100000, 512, 512
HW = H * W
BP = B * P
PMAIN = 99968
PREM = P - PMAIN
PADP = 100096
BPP = B * PADP
ROWS_P = BPP // 128
NC, NS = 2, 16
SLI = HW // NS
CNT = PADP // NS

_mesh = plsc.VectorSubcoreMesh(core_axis_name="c", subcore_axis_name="s")


def _idx_body(xam, xar, yam, yar, xbm, xbr, ybm, ybr, om, orr,
              ia_ref, ib_ref, op_ref):
    lane = lax.broadcasted_iota(jnp.int32, (1, 128), 1).reshape(128)
    valid = lane < PREM

    def side(xm, ym, xr, yr):
        main = ym[...] * W + xm[...]
        remv = yr[...] * W + xr[...]
        pieces = []
        for b in range(B):
            base = (b % 2) * HW
            pieces.append((main[b : b + 1] + base).reshape(PMAIN))
            pieces.append(jnp.where(
                valid, remv[b : b + 1].reshape(128) + base, 0))
        return jnp.concatenate(pieces, axis=0)

    ia_ref[...] = side(xam, yam, xar, yar)
    ib_ref[...] = side(xbm, ybm, xbr, ybr)
    opieces = []
    for b in range(B):
        opieces.append(om[b : b + 1].reshape(PMAIN))
        opieces.append(jnp.where(valid, orr[b : b + 1].reshape(128), 0.0))
    op_ref[...] = jnp.concatenate(opieces, axis=0)


def _in_pair():
    return (
        pl.BlockSpec((B, PMAIN), lambda i: (0, 0)),
        pl.BlockSpec((B, 128), lambda i: (0, PMAIN // 128)),
    )


_idx_call = pl.pallas_call(
    _idx_body,
    grid=(1,),
    out_specs=(
        pl.BlockSpec((BPP,), lambda i: (0,)),
        pl.BlockSpec((BPP,), lambda i: (0,)),
        pl.BlockSpec((BPP,), lambda i: (0,)),
    ),
    in_specs=[
        *_in_pair(),
        *_in_pair(),
        *_in_pair(),
        *_in_pair(),
        *_in_pair(),
    ],
    out_shape=(
        jax.ShapeDtypeStruct((BPP,), jnp.int32),
        jax.ShapeDtypeStruct((BPP,), jnp.int32),
        jax.ShapeDtypeStruct((BPP,), jnp.float32),
    ),
)


@functools.partial(
    pl.kernel,
    mesh=_mesh,
    out_type=(
        jax.ShapeDtypeStruct((BPP,), jnp.float32),
        jax.ShapeDtypeStruct((BPP,), jnp.float32),
    ),
    scratch_types=[
        pltpu.VMEM((CNT,), jnp.int32),
        pltpu.VMEM((CNT,), jnp.int32),
        pltpu.VMEM((CNT,), jnp.int32),
        pltpu.VMEM((CNT,), jnp.int32),
        pltpu.VMEM((CNT,), jnp.float32),
        pltpu.VMEM((CNT,), jnp.float32),
        pltpu.VMEM_SHARED((HW,), jnp.float32),
        pltpu.VMEM_SHARED((HW,), jnp.float32),
        pltpu.SemaphoreType.DMA,
        pltpu.SemaphoreType.DMA,
        pltpu.SemaphoreType.DMA,
    ],
)
def _sc_gather(depth, idx_a, idx_b, out_a, out_b, via0, vib0, via1, vib1,
               z_a, z_b, slab0, slab1, sem_s, sem_a, sem_b):
    c = lax.axis_index("c")
    s = lax.axis_index("s")
    slabs = (slab0, slab1)
    bufs = ((via0, vib0), (via1, vib1))

    def stage(i):
        dbase = (4 * c + i) * HW + s * SLI
        return pltpu.async_copy(
            depth.at[pl.ds(pl.multiple_of(dbase, 8), SLI)],
            slabs[i % 2].at[pl.ds(s * SLI, SLI)], sem_s)

    def ldidx(i):
        gb = pl.multiple_of((4 * c + i) * PADP + s * CNT, 8)
        va, vb = bufs[i % 2]
        pltpu.sync_copy(idx_a.at[pl.ds(gb, CNT)], va)
        pltpu.sync_copy(idx_b.at[pl.ds(gb, CNT)], vb)

    st = stage(0)
    ldidx(0)
    st.wait()
    plsc.subcore_barrier()
    for i in range(4):
        if i < 3:
            st = stage(i + 1)
        va, vb = bufs[i % 2]
        cp_a = pltpu.async_copy(slabs[i % 2].at[va], z_a, sem_a)
        cp_b = pltpu.async_copy(slabs[i % 2].at[vb], z_b, sem_b)
        if i < 3:
            ldidx(i + 1)
        gb = pl.multiple_of((4 * c + i) * PADP + s * CNT, 8)
        cp_a.wait()
        pltpu.sync_copy(z_a, out_a.at[pl.ds(gb, CNT)])
        cp_b.wait()
        pltpu.sync_copy(z_b, out_b.at[pl.ds(gb, CNT)])
        if i < 3:
            st.wait()
            plsc.subcore_barrier()


_LGRID = 2
_LROWS = ROWS_P // _LGRID


def _loss_body(za_ref, zb_ref, g_ref, o_ref):
    d = za_ref[...] - zb_ref[...]
    g = g_ref[...]
    mask = jnp.abs(g)
    loss = mask * jnp.log(1.0 + jnp.exp(-g * d)) + (1.0 - mask) * (d * d)
    part = jnp.sum(loss) / BP

    @pl.when(pl.program_id(0) == 0)
    def _():
        o_ref[0, 0] = part

    @pl.when(pl.program_id(0) != 0)
    def _():
        o_ref[0, 0] += part


_loss_call = pl.pallas_call(
    _loss_body,
    grid=(_LGRID,),
    in_specs=[
        pl.BlockSpec((_LROWS, 128), lambda i: (i, 0)),
        pl.BlockSpec((_LROWS, 128), lambda i: (i, 0)),
        pl.BlockSpec((_LROWS, 128), lambda i: (i, 0)),
    ],
    out_specs=pl.BlockSpec(memory_space=pltpu.SMEM),
    out_shape=jax.ShapeDtypeStruct((1, 1), jnp.float32),
)


def kernel(input, x_A, y_A, x_B, y_B, ordinal_relation):
    depth = input.reshape(B * H * W)
    ia, ib, ordp = _idx_call(x_A, x_A, y_A, y_A, x_B, x_B, y_B, y_B,
                             ordinal_relation, ordinal_relation)
    z_a, z_b = _sc_gather(depth, ia, ib)
    out = _loss_call(
        z_a.reshape(ROWS_P, 128),
        z_b.reshape(ROWS_P, 128),
        ordp.reshape(ROWS_P, 128),
    )
    return out.reshape(1)
